# baseline (speedup 1.0000x reference)
import functools
import math

import jax
import jax.numpy as jnp
import numpy as np
from jax import lax
from jax.experimental import pallas as pl
from jax.experimental.pallas import tpu as pltpu
from jax.experimental.pallas import tpu_sc as plsc

VOCAB = 1000000
EMBED_DIM = 64
SEQ_PROJ_DIM = 128
BATCH = 1024
SEQ_LEN = 200

NUM_IDX = BATCH * SEQ_LEN

_NC, _NS = 2, 16
_NW = _NC * _NS
_B_PER_W = NUM_IDX // _NW
_CHUNK = 640
_N_CHUNKS = _B_PER_W // _CHUNK

_ROWS_BLK = 1600


def _positional_encoding(seq_len, embed_dim):
    position = np.arange(0, seq_len, dtype=np.float32)[:, None]
    div_term = np.exp(
        np.arange(0, embed_dim, 2, dtype=np.float32) * (-math.log(10000.0) / embed_dim)
    )
    pe = np.zeros((seq_len, embed_dim), dtype=np.float32)
    pe[:, 0::2] = np.sin(position * div_term)
    pe[:, 1::2] = np.cos(position * div_term)
    return pe


def _sc_gather(table2, idx2d):
    mesh = plsc.VectorSubcoreMesh(core_axis_name="c", subcore_axis_name="s")

    @functools.partial(
        pl.kernel,
        mesh=mesh,
        out_type=jax.ShapeDtypeStruct((NUM_IDX, 2 * EMBED_DIM), jnp.float32),
        scratch_types=[
            pltpu.VMEM((_CHUNK,), jnp.int32),
            pltpu.VMEM((_CHUNK, 2 * EMBED_DIM), jnp.float32),
            pltpu.SemaphoreType.DMA,
        ],
    )
    def k(table_hbm, idx_hbm, out_hbm, idx_v, rows_v, sem):
        wid = lax.axis_index("s") * _NC + lax.axis_index("c")
        base = wid * _B_PER_W

        @pl.loop(0, _N_CHUNKS)
        def _(j):
            pltpu.sync_copy(idx_hbm.at[wid * _N_CHUNKS + j], idx_v)
            pltpu.async_copy(table_hbm.at[idx_v], rows_v, sem).wait()
            pltpu.sync_copy(rows_v, out_hbm.at[pl.ds(base + j * _CHUNK, _CHUNK)])

    return k(table2, idx2d)


def _tc_fuse(gathered, parity, pe_t, gamma, beta, W, b):
    def body(g_ref, par_ref, pe_ref, gm_ref, bt_ref, w_ref, b_ref, o_ref):
        pair = g_ref[...]
        sel = par_ref[...] > 0.5
        emb = jnp.where(sel, pair[:, EMBED_DIM:], pair[:, :EMBED_DIM])
        e = emb + pe_ref[...]
        mean = jnp.mean(e, axis=1, keepdims=True)
        c = e - mean
        var = jnp.mean(c * c, axis=1, keepdims=True)
        z = c * lax.rsqrt(var + 1e-5)
        z = z * gm_ref[...] + bt_ref[...]
        o_ref[...] = (
            jnp.dot(z, w_ref[...], preferred_element_type=jnp.float32) + b_ref[...]
        )

    return pl.pallas_call(
        body,
        grid=(NUM_IDX // _ROWS_BLK,),
        in_specs=[
            pl.BlockSpec((_ROWS_BLK, 2 * EMBED_DIM), lambda i: (i, 0)),
            pl.BlockSpec((_ROWS_BLK, 1), lambda i: (i, 0)),
            pl.BlockSpec((_ROWS_BLK, EMBED_DIM), lambda i: (0, 0)),
            pl.BlockSpec((1, EMBED_DIM), lambda i: (0, 0)),
            pl.BlockSpec((1, EMBED_DIM), lambda i: (0, 0)),
            pl.BlockSpec((EMBED_DIM, SEQ_PROJ_DIM), lambda i: (0, 0)),
            pl.BlockSpec((1, SEQ_PROJ_DIM), lambda i: (0, 0)),
        ],
        out_specs=pl.BlockSpec((_ROWS_BLK, SEQ_PROJ_DIM), lambda i: (i, 0)),
        out_shape=jax.ShapeDtypeStruct((NUM_IDX, SEQ_PROJ_DIM), jnp.float32),
    )(gathered, parity, pe_t, gamma.reshape(1, -1), beta.reshape(1, -1), W, b.reshape(1, -1))


def kernel(x, table, gamma, beta, W, b):
    idx = x.astype(jnp.int32).reshape(-1)
    idx2d = (idx >> 1).reshape(_NW * _N_CHUNKS, _CHUNK)
    parity = (idx & 1).astype(jnp.float32).reshape(NUM_IDX, 1)
    table2 = table.reshape(VOCAB // 2, 2 * EMBED_DIM)
    gathered = _sc_gather(table2, idx2d)
    pe_t = jnp.asarray(
        np.tile(_positional_encoding(SEQ_LEN, EMBED_DIM), (_ROWS_BLK // SEQ_LEN, 1))
    )
    out = _tc_fuse(gathered, parity, pe_t, gamma, beta, W, b)
    return out.reshape(BATCH, SEQ_LEN, SEQ_PROJ_DIM)


# speedup vs baseline: 1.0840x; 1.0840x over previous
import functools
import math

import jax
import jax.numpy as jnp
import numpy as np
from jax import lax
from jax.experimental import pallas as pl
from jax.experimental.pallas import tpu as pltpu
from jax.experimental.pallas import tpu_sc as plsc

VOCAB = 1000000
EMBED_DIM = 64
SEQ_PROJ_DIM = 128
BATCH = 1024
SEQ_LEN = 200

NUM_IDX = BATCH * SEQ_LEN

_NC, _NS = 2, 16
_NW = _NC * _NS
_B_PER_W = NUM_IDX // _NW
_CHUNK = 640
_N_CHUNKS = _B_PER_W // _CHUNK

_B_BLK = 16
_ROWS_BLK = _B_BLK * SEQ_LEN


def _positional_encoding(seq_len, embed_dim):
    position = np.arange(0, seq_len, dtype=np.float32)[:, None]
    div_term = np.exp(
        np.arange(0, embed_dim, 2, dtype=np.float32) * (-math.log(10000.0) / embed_dim)
    )
    pe = np.zeros((seq_len, embed_dim), dtype=np.float32)
    pe[:, 0::2] = np.sin(position * div_term)
    pe[:, 1::2] = np.cos(position * div_term)
    return pe


def _sc_gather(table2, idx2d):
    mesh = plsc.VectorSubcoreMesh(core_axis_name="c", subcore_axis_name="s")

    @functools.partial(
        pl.kernel,
        mesh=mesh,
        out_type=jax.ShapeDtypeStruct((NUM_IDX, 2 * EMBED_DIM), jnp.float32),
        scratch_types=[
            pltpu.VMEM((_CHUNK,), jnp.int32),
            pltpu.VMEM((_CHUNK, 2 * EMBED_DIM), jnp.float32),
            pltpu.SemaphoreType.DMA,
        ],
    )
    def k(table_hbm, idx_hbm, out_hbm, idx_v, rows_v, sem):
        wid = lax.axis_index("s") * _NC + lax.axis_index("c")
        base = wid * _B_PER_W

        @pl.loop(0, _N_CHUNKS)
        def _(j):
            pltpu.sync_copy(idx_hbm.at[wid * _N_CHUNKS + j], idx_v)
            pltpu.async_copy(table_hbm.at[idx_v], rows_v, sem).wait()
            pltpu.sync_copy(rows_v, out_hbm.at[pl.ds(base + j * _CHUNK, _CHUNK)])

    return k(table2, idx2d)


def _tc_fuse(gathered, parity, pe_t, gamma, beta, W, b):
    def body(g_ref, par_ref, pe_ref, gm_ref, bt_ref, w_ref, b_ref, o_ref):
        pair = g_ref[...]
        sel = par_ref[...] > 0.5
        emb = jnp.where(sel, pair[:, EMBED_DIM:], pair[:, :EMBED_DIM])
        e = emb + pe_ref[...]
        mean = jnp.mean(e, axis=1, keepdims=True)
        c = e - mean
        var = jnp.mean(c * c, axis=1, keepdims=True)
        z = c * lax.rsqrt(var + 1e-5)
        z = z * gm_ref[...] + bt_ref[...]
        res = jnp.dot(z, w_ref[...], preferred_element_type=jnp.float32) + b_ref[...]
        o_ref[...] = res.reshape(_B_BLK, SEQ_LEN, SEQ_PROJ_DIM)

    return pl.pallas_call(
        body,
        grid=(NUM_IDX // _ROWS_BLK,),
        in_specs=[
            pl.BlockSpec((_ROWS_BLK, 2 * EMBED_DIM), lambda i: (i, 0)),
            pl.BlockSpec((_ROWS_BLK, 1), lambda i: (i, 0)),
            pl.BlockSpec((_ROWS_BLK, EMBED_DIM), lambda i: (0, 0)),
            pl.BlockSpec((1, EMBED_DIM), lambda i: (0, 0)),
            pl.BlockSpec((1, EMBED_DIM), lambda i: (0, 0)),
            pl.BlockSpec((EMBED_DIM, SEQ_PROJ_DIM), lambda i: (0, 0)),
            pl.BlockSpec((1, SEQ_PROJ_DIM), lambda i: (0, 0)),
        ],
        out_specs=pl.BlockSpec((_B_BLK, SEQ_LEN, SEQ_PROJ_DIM), lambda i: (i, 0, 0)),
        out_shape=jax.ShapeDtypeStruct((BATCH, SEQ_LEN, SEQ_PROJ_DIM), jnp.float32),
    )(gathered, parity, pe_t, gamma.reshape(1, -1), beta.reshape(1, -1), W, b.reshape(1, -1))


def kernel(x, table, gamma, beta, W, b):
    idx = x.astype(jnp.int32).reshape(-1)
    idx2d = (idx >> 1).reshape(_NW * _N_CHUNKS, _CHUNK)
    parity = (idx & 1).astype(jnp.float32).reshape(NUM_IDX, 1)
    table2 = table.reshape(VOCAB // 2, 2 * EMBED_DIM)
    gathered = _sc_gather(table2, idx2d)
    pe_t = jnp.asarray(
        np.tile(_positional_encoding(SEQ_LEN, EMBED_DIM), (_ROWS_BLK // SEQ_LEN, 1))
    )
    return _tc_fuse(gathered, parity, pe_t, gamma, beta, W, b)


# speedup vs baseline: 1.2407x; 1.1445x over previous
import functools
import math

import jax
import jax.numpy as jnp
import numpy as np
from jax import lax
from jax.experimental import pallas as pl
from jax.experimental.pallas import tpu as pltpu
from jax.experimental.pallas import tpu_sc as plsc

VOCAB = 1000000
EMBED_DIM = 64
SEQ_PROJ_DIM = 128
BATCH = 1024
SEQ_LEN = 200

NUM_IDX = BATCH * SEQ_LEN

_NC, _NS = 2, 16
_NW = _NC * _NS
_B_PER_W = NUM_IDX // _NW
_CHUNK = 640
_N_CHUNKS = _B_PER_W // _CHUNK

_B_BLK = 16
_ROWS_BLK = _B_BLK * SEQ_LEN


def _positional_encoding(seq_len, embed_dim):
    position = np.arange(0, seq_len, dtype=np.float32)[:, None]
    div_term = np.exp(
        np.arange(0, embed_dim, 2, dtype=np.float32) * (-math.log(10000.0) / embed_dim)
    )
    pe = np.zeros((seq_len, embed_dim), dtype=np.float32)
    pe[:, 0::2] = np.sin(position * div_term)
    pe[:, 1::2] = np.cos(position * div_term)
    return pe


def _sc_gather(table, idx2d):
    mesh = plsc.VectorSubcoreMesh(core_axis_name="c", subcore_axis_name="s")

    @functools.partial(
        pl.kernel,
        mesh=mesh,
        out_type=jax.ShapeDtypeStruct((NUM_IDX, EMBED_DIM), jnp.float32),
        scratch_types=[
            pltpu.VMEM((_CHUNK,), jnp.int32),
            pltpu.VMEM((_CHUNK, EMBED_DIM), jnp.float32),
            pltpu.SemaphoreType.DMA,
        ],
        compiler_params=pltpu.CompilerParams(use_tc_tiling_on_sc=False),
    )
    def k(table_hbm, idx_hbm, out_hbm, idx_v, rows_v, sem):
        wid = lax.axis_index("s") * _NC + lax.axis_index("c")
        base = wid * _B_PER_W

        @pl.loop(0, _N_CHUNKS)
        def _(j):
            pltpu.sync_copy(idx_hbm.at[wid * _N_CHUNKS + j], idx_v)
            pltpu.async_copy(table_hbm.at[idx_v], rows_v, sem).wait()
            pltpu.sync_copy(rows_v, out_hbm.at[pl.ds(base + j * _CHUNK, _CHUNK)])

    return k(table, idx2d)


def _tc_fuse(gathered, pe_t, gamma, beta, W, b):
    def body(g_ref, pe_ref, gm_ref, bt_ref, w_ref, b_ref, o_ref):
        e = g_ref[...] + pe_ref[...]
        mean = jnp.mean(e, axis=1, keepdims=True)
        c = e - mean
        var = jnp.mean(c * c, axis=1, keepdims=True)
        z = c * lax.rsqrt(var + 1e-5)
        z = z * gm_ref[...] + bt_ref[...]
        res = jnp.dot(z, w_ref[...], preferred_element_type=jnp.float32) + b_ref[...]
        o_ref[...] = res.reshape(_B_BLK, SEQ_LEN, SEQ_PROJ_DIM)

    return pl.pallas_call(
        body,
        grid=(NUM_IDX // _ROWS_BLK,),
        in_specs=[
            pl.BlockSpec((_ROWS_BLK, EMBED_DIM), lambda i: (i, 0)),
            pl.BlockSpec((_ROWS_BLK, EMBED_DIM), lambda i: (0, 0)),
            pl.BlockSpec((1, EMBED_DIM), lambda i: (0, 0)),
            pl.BlockSpec((1, EMBED_DIM), lambda i: (0, 0)),
            pl.BlockSpec((EMBED_DIM, SEQ_PROJ_DIM), lambda i: (0, 0)),
            pl.BlockSpec((1, SEQ_PROJ_DIM), lambda i: (0, 0)),
        ],
        out_specs=pl.BlockSpec((_B_BLK, SEQ_LEN, SEQ_PROJ_DIM), lambda i: (i, 0, 0)),
        out_shape=jax.ShapeDtypeStruct((BATCH, SEQ_LEN, SEQ_PROJ_DIM), jnp.float32),
    )(gathered, pe_t, gamma.reshape(1, -1), beta.reshape(1, -1), W, b.reshape(1, -1))


def kernel(x, table, gamma, beta, W, b):
    idx2d = x.astype(jnp.int32).reshape(_NW * _N_CHUNKS, _CHUNK)
    gathered = _sc_gather(table, idx2d)
    pe_t = jnp.asarray(
        np.tile(_positional_encoding(SEQ_LEN, EMBED_DIM), (_ROWS_BLK // SEQ_LEN, 1))
    )
    return _tc_fuse(gathered, pe_t, gamma, beta, W, b)
